# Initial kernel scaffold; baseline (speedup 1.0000x reference)
import jax
import jax.numpy as jnp
from jax.experimental import pallas as pl


def kernel(x, edge_index, edge_weight, W, bias, alpha):
    raise NotImplementedError("write your pallas kernel here")


# speedup vs baseline: 72.2644x; 72.2644x over previous
import functools

import jax
import jax.numpy as jnp
from jax import lax
from jax.experimental import pallas as pl
from jax.experimental.pallas import tpu as pltpu
from jax.experimental.pallas import tpu_sc as plsc

_A = 1.0
_B = 1.0

_NC = 2
_NSUB = 16
_CHUNK = 128
_RING = 4


def _largest_divisor_le(n, cap):
    for d in range(min(cap, n), 0, -1):
        if n % d == 0:
            return d
    return 1


@functools.lru_cache(maxsize=None)
def _make_sc_spmm(N_acc, D, n_chunks):
    rows_per = N_acc // _NSUB
    zslab = _largest_divisor_le(rows_per, _CHUNK)
    nvec = D // 16

    mesh = plsc.VectorSubcoreMesh(core_axis_name="c", subcore_axis_name="s",
                                  num_cores=_NC)

    @functools.partial(
        pl.kernel,
        out_type=jax.ShapeDtypeStruct((_NC, N_acc, D), jnp.float32),
        mesh=mesh,
        scratch_types=[
            pltpu.VMEM((_RING, _CHUNK), jnp.int32),
            pltpu.VMEM((_RING, _CHUNK), jnp.int32),
            pltpu.VMEM((_RING, _CHUNK), jnp.float32),
            pltpu.VMEM((_CHUNK, D), jnp.float32),
            pltpu.VMEM((_CHUNK, D), jnp.float32),
            pltpu.VMEM_SHARED((N_acc, D), jnp.float32),
            pltpu.SemaphoreType.DMA,
            pltpu.SemaphoreType.DMA,
            pltpu.SemaphoreType.DMA,
            pltpu.SemaphoreType.DMA,
            pltpu.SemaphoreType.DMA,
            pltpu.SemaphoreType.DMA,
        ],
    )
    def sc_spmm(p_hbm, col_hbm, row_hbm, w_hbm, part,
                col_r, row_r, w_r, rows0, rows1, acc,
                sf0, sf1, sf2, sf3, sg0, sg1):
        cid = lax.axis_index("c")
        sid = lax.axis_index("s")
        fsems = (sf0, sf1, sf2, sf3)
        gsems = (sg0, sg1)
        bufs = (rows0, rows1)

        def fetch(chunk, slot):
            pltpu.async_copy(col_hbm.at[cid, sid, chunk], col_r.at[slot],
                             fsems[slot])
            pltpu.async_copy(row_hbm.at[cid, sid, chunk], row_r.at[slot],
                             fsems[slot])
            pltpu.async_copy(w_hbm.at[cid, sid, chunk], w_r.at[slot],
                             fsems[slot])

        def drain_fetch(slot):
            for _ in range(3):
                pltpu.make_async_copy(col_hbm.at[cid, sid, 0],
                                      col_r.at[slot], fsems[slot]).wait()

        def gather(slot, b2):
            pltpu.async_copy(p_hbm.at[col_r.at[slot]], bufs[b2], gsems[b2])

        def drain_gather(b2):
            pltpu.make_async_copy(p_hbm.at[pl.ds(0, _CHUNK)], bufs[b2],
                                  gsems[b2]).wait()

        def scale(slot, buf):
            def gbody(g, carry):
                w16 = w_r[slot, pl.ds(g * 16, 16)]
                for i in range(16):
                    ws = w16[i]
                    e = g * 16 + i
                    for v in range(nvec):
                        sl = pl.ds(v * 16, 16)
                        buf[e, sl] = buf[e, sl] * ws
                return carry
            lax.fori_loop(0, _CHUNK // 16, gbody, 0)

        def zbody(i, carry):
            for v in range(nvec):
                rows0[i, pl.ds(v * 16, 16)] = jnp.zeros((16,), jnp.float32)
            return carry
        lax.fori_loop(0, _CHUNK, zbody, 0)
        base = sid * rows_per
        for i in range(rows_per // zslab):
            pltpu.sync_copy(rows0.at[pl.ds(0, zslab)],
                            acc.at[pl.ds(base + i * zslab, zslab)])
        plsc.subcore_barrier()

        for s in range(_RING):
            fetch(s, s)
        for j in range(2):
            drain_fetch(j)
            gather(j, j)

        def quad_body(jq, carry):
            for bb in range(4):
                j = jq * 4 + bb
                b2 = bb % 2
                buf = bufs[b2]
                drain_gather(b2)
                scale(bb, buf)
                pltpu.sync_copy(buf, acc.at[row_r.at[bb]], add=True)

                @pl.when(j + _RING < n_chunks)
                def _():
                    fetch(j + _RING, bb)

                @pl.when(j + 2 < n_chunks)
                def _():
                    drain_fetch((bb + 2) % 4)
                    gather((bb + 2) % 4, b2)
            return carry
        lax.fori_loop(0, n_chunks // 4, quad_body, 0)
        plsc.subcore_barrier()

        pltpu.sync_copy(acc.at[pl.ds(base, rows_per)],
                        part.at[cid, pl.ds(base, rows_per)])

    return sc_spmm


def _h0_kernel(x, w_flat, bias_f, alpha0):
    N, D = x.shape
    M = w_flat.shape[1]
    blk = _largest_divisor_le(N, 1024)

    def body(x_ref, w_ref, b_ref, a_ref, h_ref, z_ref):
        h = jnp.dot(x_ref[...], w_ref[...],
                    preferred_element_type=jnp.float32) + b_ref[...]
        h_ref[...] = h
        z_ref[...] = a_ref[...] * h

    return pl.pallas_call(
        body,
        grid=(N // blk,),
        in_specs=[
            pl.BlockSpec((blk, D), lambda i: (i, 0)),
            pl.BlockSpec((D, M), lambda i: (0, 0)),
            pl.BlockSpec((1, M), lambda i: (0, 0)),
            pl.BlockSpec((1, M), lambda i: (0, 0)),
        ],
        out_specs=[
            pl.BlockSpec((blk, M), lambda i: (i, 0)),
            pl.BlockSpec((blk, M), lambda i: (i, 0)),
        ],
        out_shape=[
            jax.ShapeDtypeStruct((N, M), jnp.float32),
            jax.ShapeDtypeStruct((N, M), jnp.float32),
        ],
    )(x, w_flat, bias_f, alpha0)


def _combine_kernel(part, pp, z, ak, theta, thd):
    N, M = z.shape
    blk = _largest_divisor_le(N, 1024)
    use_pp = thd != 0.0

    def body(p_ref, pp_ref, z_ref, a_ref, pn_ref, zo_ref):
        s = theta * (p_ref[0] + p_ref[1])
        if use_pp:
            s = s - thd * pp_ref[...]
        pn_ref[...] = s
        zo_ref[...] = z_ref[...] + a_ref[...] * s

    return pl.pallas_call(
        body,
        grid=(N // blk,),
        in_specs=[
            pl.BlockSpec((_NC, blk, M), lambda i: (0, i, 0)),
            pl.BlockSpec((blk, M), lambda i: (i, 0)),
            pl.BlockSpec((blk, M), lambda i: (i, 0)),
            pl.BlockSpec((1, M), lambda i: (0, 0)),
        ],
        out_specs=[
            pl.BlockSpec((blk, M), lambda i: (i, 0)),
            pl.BlockSpec((blk, M), lambda i: (i, 0)),
        ],
        out_shape=[
            jax.ShapeDtypeStruct((N, M), jnp.float32),
            jax.ShapeDtypeStruct((N, M), jnp.float32),
        ],
    )(part, pp, z, ak)


def kernel(x, edge_index, edge_weight, W, bias, alpha):
    N, D = x.shape
    T, _, C = W.shape
    K = alpha.shape[1] - 1
    M = T * C
    E = edge_weight.shape[0]
    a, b = _A, _B

    w_flat = jnp.transpose(W, (1, 0, 2)).reshape(D, M)
    bias_f = bias.reshape(1, M)
    alpha_f = jnp.transpose(alpha, (1, 0, 2)).reshape(K + 1, M)

    nw = _NC * _NSUB
    n_chunks = 4 * -(-E // (nw * _CHUNK * 4))
    e_pad = nw * _CHUNK * n_chunks
    shp = (_NC, _NSUB, n_chunks, _CHUNK)
    col3 = jnp.pad(edge_index[1], (0, e_pad - E)).reshape(shp)
    row3 = jnp.pad(edge_index[0], (0, e_pad - E)).reshape(shp)
    w3 = jnp.pad(edge_weight, (0, e_pad - E)).reshape(shp)

    n_acc = -(-N // (_NSUB * 8)) * _NSUB * 8
    sc_spmm = _make_sc_spmm(n_acc, M, n_chunks)

    h0, z = _h0_kernel(x, w_flat, bias_f, alpha_f[0:1])

    part = sc_spmm(h0, col3, row3, w3)
    p_prev = h0
    p_curr, z = _combine_kernel(part, h0, z, alpha_f[1:2],
                                (a + b + 2.0) / 2.0, 0.0)
    for k in range(2, K + 1):
        th = (2 * k + a + b) * (2 * k + a + b - 1) / (2 * k * (k + a + b))
        thd = ((k + a - 1) * (k + b - 1) * (2 * k + a + b)
               / (k * (k + a + b) * (2 * k + a + b - 2)))
        part = sc_spmm(p_curr, col3, row3, w3)
        p_next, z = _combine_kernel(part, p_prev, z,
                                    alpha_f[k:k + 1], th, thd)
        p_prev, p_curr = p_curr, p_next

    return jnp.transpose(z.reshape(N, T, C), (1, 0, 2))


# speedup vs baseline: 242.5976x; 3.3571x over previous
import functools

import jax
import jax.numpy as jnp
from jax import lax
from jax.experimental import pallas as pl
from jax.experimental.pallas import tpu as pltpu
from jax.experimental.pallas import tpu_sc as plsc

_A = 1.0
_B = 1.0

_NC = 2
_NSUB = 16
_CHUNK = 128
_RING = 4


def _largest_divisor_le(n, cap):
    for d in range(min(cap, n), 0, -1):
        if n % d == 0:
            return d
    return 1


@functools.lru_cache(maxsize=None)
def _make_sc_spmm(N_acc, D, n_chunks):
    rows_per = N_acc // _NSUB
    zslab = _largest_divisor_le(rows_per, _CHUNK)
    nvec = D // 16

    mesh = plsc.VectorSubcoreMesh(core_axis_name="c", subcore_axis_name="s",
                                  num_cores=_NC)

    @functools.partial(
        pl.kernel,
        out_type=jax.ShapeDtypeStruct((_NC, N_acc, D), jnp.float32),
        mesh=mesh,
        scratch_types=[
            pltpu.VMEM((_RING, _CHUNK), jnp.int32),
            pltpu.VMEM((_RING, _CHUNK), jnp.int32),
            pltpu.VMEM((_RING, _CHUNK), jnp.float32),
            pltpu.VMEM((_CHUNK, D), jnp.float32),
            pltpu.VMEM((_CHUNK, D), jnp.float32),
            pltpu.VMEM_SHARED((N_acc, D), jnp.float32),
            pltpu.SemaphoreType.DMA,
            pltpu.SemaphoreType.DMA,
            pltpu.SemaphoreType.DMA,
            pltpu.SemaphoreType.DMA,
            pltpu.SemaphoreType.DMA,
            pltpu.SemaphoreType.DMA,
        ],
    )
    def sc_spmm(p_hbm, col_hbm, row_hbm, w_hbm, part,
                col_r, row_r, w_r, rows0, rows1, acc,
                sf0, sf1, sf2, sf3, sg0, sg1):
        cid = lax.axis_index("c")
        sid = lax.axis_index("s")
        fsems = (sf0, sf1, sf2, sf3)
        gsems = (sg0, sg1)
        bufs = (rows0, rows1)

        def fetch(chunk, slot):
            pltpu.async_copy(col_hbm.at[cid, sid, chunk], col_r.at[slot],
                             fsems[slot])
            pltpu.async_copy(row_hbm.at[cid, sid, chunk], row_r.at[slot],
                             fsems[slot])
            pltpu.async_copy(w_hbm.at[cid, sid, chunk], w_r.at[slot],
                             fsems[slot])

        def drain_fetch(slot):
            for _ in range(3):
                pltpu.make_async_copy(col_hbm.at[cid, sid, 0],
                                      col_r.at[slot], fsems[slot]).wait()

        def gather(slot, b2):
            pltpu.async_copy(p_hbm.at[col_r.at[slot]], bufs[b2], gsems[b2])

        def drain_gather(b2):
            pltpu.make_async_copy(p_hbm.at[pl.ds(0, _CHUNK)], bufs[b2],
                                  gsems[b2]).wait()

        def scale(slot, buf):
            def gbody(g, carry):
                w16 = w_r[slot, pl.ds(g * 16, 16)]
                for i in range(16):
                    ws = w16[i]
                    e = g * 16 + i
                    for v in range(nvec):
                        sl = pl.ds(v * 16, 16)
                        buf[e, sl] = buf[e, sl] * ws
                return carry
            lax.fori_loop(0, _CHUNK // 16, gbody, 0)

        def zbody(i, carry):
            for v in range(nvec):
                rows0[i, pl.ds(v * 16, 16)] = jnp.zeros((16,), jnp.float32)
            return carry
        lax.fori_loop(0, _CHUNK, zbody, 0)
        base = sid * rows_per
        for i in range(rows_per // zslab):
            pltpu.sync_copy(rows0.at[pl.ds(0, zslab)],
                            acc.at[pl.ds(base + i * zslab, zslab)])
        plsc.subcore_barrier()

        for s in range(_RING):
            fetch(s, s)
        for j in range(2):
            drain_fetch(j)
            gather(j, j)

        def quad_body(jq, carry):
            for bb in range(4):
                j = jq * 4 + bb
                b2 = bb % 2
                buf = bufs[b2]
                drain_gather(b2)
                scale(bb, buf)
                pltpu.sync_copy(buf, acc.at[row_r.at[bb]], add=True)

                @pl.when(j + _RING < n_chunks)
                def _():
                    fetch(j + _RING, bb)

                @pl.when(j + 2 < n_chunks)
                def _():
                    drain_fetch((bb + 2) % 4)
                    gather((bb + 2) % 4, b2)
            return carry
        lax.fori_loop(0, n_chunks // 4, quad_body, 0)
        plsc.subcore_barrier()

        pltpu.sync_copy(acc.at[pl.ds(base, rows_per)],
                        part.at[cid, pl.ds(base, rows_per)])

    return sc_spmm


def _h0_kernel(x, w_flat, bias_f, alpha0):
    N, D = x.shape
    M = w_flat.shape[1]
    blk = _largest_divisor_le(N, 1024)

    def body(x_ref, w_ref, b_ref, a_ref, h_ref, z_ref):
        h = jnp.dot(x_ref[...], w_ref[...],
                    preferred_element_type=jnp.float32) + b_ref[...]
        h_ref[...] = h
        z_ref[...] = a_ref[...] * h

    return pl.pallas_call(
        body,
        grid=(N // blk,),
        in_specs=[
            pl.BlockSpec((blk, D), lambda i: (i, 0)),
            pl.BlockSpec((D, M), lambda i: (0, 0)),
            pl.BlockSpec((1, M), lambda i: (0, 0)),
            pl.BlockSpec((1, M), lambda i: (0, 0)),
        ],
        out_specs=[
            pl.BlockSpec((blk, M), lambda i: (i, 0)),
            pl.BlockSpec((blk, M), lambda i: (i, 0)),
        ],
        out_shape=[
            jax.ShapeDtypeStruct((N, M), jnp.float32),
            jax.ShapeDtypeStruct((N, M), jnp.float32),
        ],
    )(x, w_flat, bias_f, alpha0)


def _combine_kernel(part, pp, z, ak, theta, thd):
    N, M = z.shape
    blk = _largest_divisor_le(N, 1024)
    use_pp = thd != 0.0

    def body(p_ref, pp_ref, z_ref, a_ref, pn_ref, zo_ref):
        s = theta * (p_ref[0] + p_ref[1])
        if use_pp:
            s = s - thd * pp_ref[...]
        pn_ref[...] = s
        zo_ref[...] = z_ref[...] + a_ref[...] * s

    return pl.pallas_call(
        body,
        grid=(N // blk,),
        in_specs=[
            pl.BlockSpec((_NC, blk, M), lambda i: (0, i, 0)),
            pl.BlockSpec((blk, M), lambda i: (i, 0)),
            pl.BlockSpec((blk, M), lambda i: (i, 0)),
            pl.BlockSpec((1, M), lambda i: (0, 0)),
        ],
        out_specs=[
            pl.BlockSpec((blk, M), lambda i: (i, 0)),
            pl.BlockSpec((blk, M), lambda i: (i, 0)),
        ],
        out_shape=[
            jax.ShapeDtypeStruct((N, M), jnp.float32),
            jax.ShapeDtypeStruct((N, M), jnp.float32),
        ],
    )(part, pp, z, ak)


def kernel(x, edge_index, edge_weight, W, bias, alpha):
    N, D = x.shape
    T, _, C = W.shape
    K = alpha.shape[1] - 1
    M = T * C
    E = edge_weight.shape[0]
    a, b = _A, _B

    w_flat = jnp.transpose(W, (1, 0, 2)).reshape(D, M)
    bias_f = bias.reshape(1, M)
    alpha_f = jnp.transpose(alpha, (1, 0, 2)).reshape(K + 1, M)

    nw = _NC * _NSUB
    n_chunks = 4 * -(-E // (nw * _CHUNK * 4))
    e_pad = nw * _CHUNK * n_chunks
    n_acc = -(-N // (_NSUB * 8)) * _NSUB * 8
    shp = (_NC, _NSUB, n_chunks, _CHUNK)
    npad = e_pad - E
    seq = jnp.arange(npad, dtype=jnp.int32)
    spare = n_acc - N
    pad_row = N + seq % spare if spare > 0 else seq % N
    pad_col = seq % N
    col3 = jnp.concatenate([edge_index[1], pad_col]).reshape(shp)
    row3 = jnp.concatenate([edge_index[0], pad_row]).reshape(shp)
    w3 = jnp.pad(edge_weight, (0, npad)).reshape(shp)
    sc_spmm = _make_sc_spmm(n_acc, M, n_chunks)

    h0, z = _h0_kernel(x, w_flat, bias_f, alpha_f[0:1])

    part = sc_spmm(h0, col3, row3, w3)
    p_prev = h0
    p_curr, z = _combine_kernel(part, h0, z, alpha_f[1:2],
                                (a + b + 2.0) / 2.0, 0.0)
    for k in range(2, K + 1):
        th = (2 * k + a + b) * (2 * k + a + b - 1) / (2 * k * (k + a + b))
        thd = ((k + a - 1) * (k + b - 1) * (2 * k + a + b)
               / (k * (k + a + b) * (2 * k + a + b - 2)))
        part = sc_spmm(p_curr, col3, row3, w3)
        p_next, z = _combine_kernel(part, p_prev, z,
                                    alpha_f[k:k + 1], th, thd)
        p_prev, p_curr = p_curr, p_next

    return jnp.transpose(z.reshape(N, T, C), (1, 0, 2))
